# Initial kernel scaffold; baseline (speedup 1.0000x reference)
import jax
import jax.numpy as jnp
from jax.experimental import pallas as pl


def kernel(x, edge_index, W1, b1, W2, b2):
    raise NotImplementedError("write your pallas kernel here")


# speedup vs baseline: 2.6081x; 2.6081x over previous
import functools

import jax
import jax.numpy as jnp
from jax import lax
from jax.experimental import pallas as pl
from jax.experimental.pallas import tpu as pltpu
from jax.experimental.pallas import tpu_sc as plsc

NC = 2
NS = 16
RNB = 632
ACCR = 640
PK = 1024
GCH = 128
EB = 10000
FW = 1920
FWR = FW // 8
STRR = FWR + 24
CAPR = (80000 // FW + 1) * FWR + STRR


def _sc_mesh():
    return plsc.VectorSubcoreMesh(core_axis_name="c", subcore_axis_name="s")


def _per_tile_node_slice(s, fn):
    @pl.when(s < NS - 1)
    def _():
        fn(pl.multiple_of(s * RNB, 8), RNB)

    @pl.when(s == NS - 1)
    def _():
        fn((NS - 1) * RNB, 10000 - (NS - 1) * RNB)


def _iota16():
    return lax.iota(jnp.int32, 16)


def _compact(src, dst, N, E):
    EH = E // NC

    @functools.partial(
        pl.kernel,
        out_type=[
            jax.ShapeDtypeStruct((NC, NS, CAPR, 128), jnp.int32),
            jax.ShapeDtypeStruct((NC, NS, 16), jnp.int32),
            jax.ShapeDtypeStruct((NC, NS, ACCR // 8, 128), jnp.float32),
        ],
        mesh=_sc_mesh(),
        scratch_types=[
            pltpu.VMEM((EB,), jnp.int32),
            pltpu.VMEM((EB,), jnp.int32),
            pltpu.VMEM((STRR, 128), jnp.int32),
            pltpu.VMEM((16,), jnp.int32),
            pltpu.VMEM((ACCR // 8, 128), jnp.float32),
        ],
    )
    def ck(src_h, dst_h, lists_h, counts_h, degp_h, sv, dv, stg, cb, deg):
        c = lax.axis_index("c")
        s = lax.axis_index("s")
        base = s * RNB
        rs = jnp.where(s == NS - 1, 10000 - (NS - 1) * RNB, RNB)
        iota = _iota16()
        e0 = jnp.where(iota == 0, 1.0, 0.0).astype(jnp.float32)

        def zdeg(i, carry):
            for g in range(8):
                deg[i, pl.ds(g * 16, 16)] = jnp.zeros((16,), jnp.float32)
            return carry

        lax.fori_loop(0, ACCR // 8, zdeg, 0)

        def put(pos, val):
            stg[pos // 8, pl.ds((pos % 8) * 16, 16)] = (
                jnp.zeros((16,), jnp.int32) + val
            )

        def block(b, carry):
            off, nf = carry
            pltpu.sync_copy(src_h.at[pl.ds(c * EH + b * EB, EB)], sv)
            pltpu.sync_copy(dst_h.at[pl.ds(c * EH + b * EB, EB)], dv)

            def chunk(j, carry):
                off, nf = carry
                d = dv[pl.ds(j * 16, 16)]
                sr = sv[pl.ds(j * 16, 16)]
                dl = d - base
                msk = (dl >= 0) & (dl < rs)
                ones = jnp.where(msk, 1, 0).astype(jnp.int32)
                incl = ones
                for step in (1, 2, 4, 8):
                    shifted = incl[jnp.maximum(iota - step, 0)]
                    incl = incl + jnp.where(iota >= step, shifted, 0)
                excl = incl - ones
                total = incl[15]
                entry = sr * PK + dl
                wbase = off - nf * FW

                @pl.when(total > 0)
                def _():
                    for lane in range(16):
                        @pl.when(ones[lane] == 1)
                        def _():
                            put(wbase + excl[lane], entry[lane])
                            dll = dl[lane]
                            deg[dll // 8, pl.ds((dll % 8) * 16, 16)] = (
                                deg[dll // 8, pl.ds((dll % 8) * 16, 16)] + e0
                            )

                off = off + total
                nf2 = nf + jnp.where(off - nf * FW >= FW, 1, 0)

                @pl.when(nf2 > nf)
                def _():
                    pltpu.sync_copy(
                        stg.at[pl.ds(0, FWR)],
                        lists_h.at[c, s, pl.ds(nf * FWR, FWR)],
                    )
                    for r in range(STRR - FWR):
                        stg[r, :] = stg[FWR + r, :]

                return (off, nf2)

            return lax.fori_loop(0, EB // 16, chunk, (off, nf))

        n, nf = lax.fori_loop(0, EH // EB, block, (0, 0))

        wbase = n - nf * FW

        def pad(k, carry):
            put(wbase + k, ACCR - 1)
            return carry

        lax.fori_loop(0, GCH, pad, 0)
        pltpu.sync_copy(
            stg.at[pl.ds(0, FWR)], lists_h.at[c, s, pl.ds(nf * FWR, FWR)]
        )
        pltpu.sync_copy(
            stg.at[pl.ds(FWR, STRR - FWR)],
            lists_h.at[c, s, pl.ds(nf * FWR + FWR, STRR - FWR)],
        )

        cb[...] = jnp.full((16,), 0, jnp.int32) + n
        pltpu.sync_copy(cb, counts_h.at[c, s])
        pltpu.sync_copy(deg, degp_h.at[c, s])

    return ck(src, dst)


def _aggregate(ycat, lists, counts, N):
    @functools.partial(
        pl.kernel,
        out_type=jax.ShapeDtypeStruct((NC, N, 128), jnp.float32),
        mesh=_sc_mesh(),
        scratch_types=[
            pltpu.VMEM((GCH // 8, 128), jnp.int32),
            pltpu.VMEM((GCH,), jnp.int32),
            pltpu.VMEM((16,), jnp.int32),
            pltpu.VMEM((GCH, 128), jnp.float32),
            pltpu.VMEM((ACCR, 128), jnp.float32),
            pltpu.SemaphoreType.DMA,
        ],
    )
    def ak(ycat_h, lists_h, counts_h, out_h, env, gidx, cb, rows, acc, sem):
        c = lax.axis_index("c")
        s = lax.axis_index("s")

        _per_tile_node_slice(
            s,
            lambda start, size: pltpu.sync_copy(
                ycat_h.at[pl.ds(c * N + start, size)], acc.at[pl.ds(0, size)]
            ),
        )

        iota = _iota16()

        def ent(i):
            return env[i // 8, pl.ds((i % 8) * 16, 16)][0]

        for ch in range(NC):
            pltpu.sync_copy(counts_h.at[ch, s], cb)
            n = cb[...][0]

            def chunk(j, carry):
                pltpu.sync_copy(
                    lists_h.at[ch, s, pl.ds(j * (GCH // 8), GCH // 8)], env
                )
                for k in range(GCH // 16):
                    gvec = jnp.zeros((16,), jnp.int32)
                    for lane in range(16):
                        gvec = jnp.where(
                            iota == lane, ent(k * 16 + lane), gvec
                        )
                    gidx[pl.ds(k * 16, 16)] = (
                        lax.shift_right_logical(gvec, 10) + c * N
                    )
                for k in range(GCH // 16):
                    pltpu.make_async_copy(
                        ycat_h.at[gidx.at[pl.ds(k * 16, 16)]],
                        rows.at[pl.ds(k * 16, 16)],
                        sem,
                    ).start()
                for k in range(GCH // 16):
                    pltpu.make_async_copy(
                        ycat_h.at[gidx.at[pl.ds(k * 16, 16)]],
                        rows.at[pl.ds(k * 16, 16)],
                        sem,
                    ).wait()

                def group(k, carry2):
                    for lane in range(16):
                        dl = env[
                            2 * k + lane // 8, pl.ds((lane % 8) * 16, 16)
                        ][0] & (PK - 1)
                        for q in range(8):
                            acc[dl, pl.ds(q * 16, 16)] = (
                                acc[dl, pl.ds(q * 16, 16)]
                                + rows[k * 16 + lane, pl.ds(q * 16, 16)]
                            )
                    return carry2

                lax.fori_loop(0, GCH // 16, group, 0)
                return carry

            trip = (n + GCH - 1) // GCH
            lax.fori_loop(0, trip, chunk, 0)

        _per_tile_node_slice(
            s,
            lambda start, size: pltpu.sync_copy(
                acc.at[pl.ds(0, size)], out_h.at[c, pl.ds(start, size)]
            ),
        )

    return ak(ycat, lists, counts)


def _degcombine(degp, N):
    NP = NS * RNB

    def body(dp_ref, out_ref):
        out_ref[...] = dp_ref[0, 0, :RNB, :] + dp_ref[1, 0, :RNB, :]

    return pl.pallas_call(
        body,
        grid=(NS,),
        in_specs=[pl.BlockSpec((NC, 1, ACCR, 16), lambda s: (0, s, 0, 0))],
        out_specs=pl.BlockSpec((RNB, 16), lambda s: (s, 0)),
        out_shape=jax.ShapeDtypeStruct((NP, 16), jnp.float32),
    )(degp)


def _scale(x, degw, N):
    R = 1000
    nb = N // R

    def body(deg_ref, x_ref, out_ref):
        dis = lax.rsqrt(deg_ref[:, 0:1] + 1.0)
        out_ref[0] = x_ref[:, :128] * dis
        out_ref[1] = x_ref[:, 128:] * dis

    return pl.pallas_call(
        body,
        grid=(nb,),
        in_specs=[
            pl.BlockSpec((R, 16), lambda i: (i, 0)),
            pl.BlockSpec((R, 256), lambda i: (i, 0)),
        ],
        out_specs=pl.BlockSpec((NC, R, 128), lambda i: (0, i, 0)),
        out_shape=jax.ShapeDtypeStruct((NC, N, 128), jnp.float32),
    )(degw, x)


def _mlp(acc1, degw, W1, b1, W2, N):
    R = 400
    nb = N // R
    d_in, d_hid = W1.shape
    d_out = W2.shape[1]

    def body(deg_ref, acc_ref, W1_ref, b1_ref, W2_ref, out_ref):
        dis = lax.rsqrt(deg_ref[:, 0:1] + 1.0)
        a = jnp.concatenate([acc_ref[0], acc_ref[1]], axis=1) * dis
        h = jnp.dot(a, W1_ref[...], preferred_element_type=jnp.float32) + b1_ref[...]
        h = jnp.maximum(h, 0.0)
        y2 = jnp.dot(h, W2_ref[...], preferred_element_type=jnp.float32) * dis
        out_ref[0] = y2[:, :128]
        out_ref[1] = y2[:, 128:]

    return pl.pallas_call(
        body,
        grid=(nb,),
        in_specs=[
            pl.BlockSpec((R, 16), lambda i: (i, 0)),
            pl.BlockSpec((NC, R, 128), lambda i: (0, i, 0)),
            pl.BlockSpec((d_in, d_hid), lambda i: (0, 0)),
            pl.BlockSpec((1, d_hid), lambda i: (0, 0)),
            pl.BlockSpec((d_hid, d_out), lambda i: (0, 0)),
        ],
        out_specs=pl.BlockSpec((NC, R, 128), lambda i: (0, i, 0)),
        out_shape=jax.ShapeDtypeStruct((NC, N, 128), jnp.float32),
    )(degw, acc1, W1, b1.reshape(1, d_hid), W2)


def _final(acc2, degw, b2, N):
    R = 1000
    nb = N // R
    d_out = b2.shape[0]

    def body(deg_ref, acc_ref, b2_ref, out_ref):
        dis = lax.rsqrt(deg_ref[:, 0:1] + 1.0)
        out_ref[...] = (
            jnp.concatenate([acc_ref[0], acc_ref[1]], axis=1) * dis + b2_ref[...]
        )

    return pl.pallas_call(
        body,
        grid=(nb,),
        in_specs=[
            pl.BlockSpec((R, 16), lambda i: (i, 0)),
            pl.BlockSpec((NC, R, 128), lambda i: (0, i, 0)),
            pl.BlockSpec((1, d_out), lambda i: (0, 0)),
        ],
        out_specs=pl.BlockSpec((R, d_out), lambda i: (i, 0)),
        out_shape=jax.ShapeDtypeStruct((N, d_out), jnp.float32),
    )(degw, acc2, b2.reshape(1, d_out))


def kernel(x, edge_index, W1, b1, W2, b2):
    N = x.shape[0]
    E = edge_index.shape[1]
    src = edge_index[0].astype(jnp.int32)
    dst = edge_index[1].astype(jnp.int32)

    lists, counts, degp = _compact(src, dst, N, E)
    degw = _degcombine(degp.reshape(NC, NS, ACCR, 16), N)[:N]
    ycat = _scale(x, degw, N).reshape(NC * N, 128)
    acc1 = _aggregate(ycat, lists, counts, N)
    y2 = _mlp(acc1, degw, W1, b1, W2, N).reshape(NC * N, 128)
    acc2 = _aggregate(y2, lists, counts, N)
    return _final(acc2, degw, b2, N)
